# baseline (speedup 1.0000x reference)
import jax
import jax.numpy as jnp
from jax.experimental import pallas as pl
from jax.experimental.pallas import tpu as pltpu

_B, _H, _W, _C = 32, 14, 14, 512
_KM_ITERS = 11
_SPLIT_H = 7
_IN_H = 2
_PH = _H // _IN_H


def _fused_body(x_ref, perm_ref, c0_ref, c1_ref,
                cm_ref, bv_ref, bh_ref, mask_ref, xs_ref):
    i = pl.program_id(0)

    @pl.when(i == 0)
    def _init():
        cm_ref[...] = jnp.full((_W, _B, _C), -jnp.inf, jnp.float32)
        bv_ref[...] = jnp.full((_B, _C), -jnp.inf, jnp.float32)
        bh_ref[...] = jnp.zeros((_B, _C), jnp.float32)

    @pl.when(i < _PH)
    def _scan():
        x = x_ref[...]
        xs_ref[pl.ds(i * _IN_H, _IN_H)] = x
        cm_ref[...] = jnp.maximum(cm_ref[...], jnp.max(x, axis=0))
        rm = jnp.max(x, axis=1)
        bv = bv_ref[...]
        bh = bh_ref[...]
        for k in range(_IN_H):
            upd = rm[k] > bv
            bv = jnp.where(upd, rm[k], bv)
            bh = jnp.where(upd, (i * _IN_H + k).astype(jnp.float32), bh)
        bv_ref[...] = bv
        bh_ref[...] = bh

    @pl.when(i == _PH - 1)
    def _kmeans():
        cm = cm_ref[...]
        best = cm[0]
        aw = jnp.zeros((_B, _C), jnp.float32)
        for w in range(1, _W):
            upd = cm[w] > best
            best = jnp.where(upd, cm[w], best)
            aw = jnp.where(upd, jnp.float32(w), aw)
        px = aw
        py = bh_ref[...]

        P = perm_ref[...]
        cx = jnp.dot(P, px[:, 0:2], preferred_element_type=jnp.float32)
        cy = jnp.dot(P, py[:, 0:2], preferred_element_type=jnp.float32)
        c0x, c1x = cx[:, 0:1], cx[:, 1:2]
        c0y, c1y = cy[:, 0:1], cy[:, 1:2]
        m1 = jnp.zeros((_B, _C), jnp.float32)
        for _ in range(_KM_ITERS):
            d0 = (px - c0x) ** 2 + (py - c0y) ** 2
            d1 = (px - c1x) ** 2 + (py - c1y) ** 2
            m1 = (d1 < d0).astype(jnp.float32)
            m0 = 1.0 - m1
            s1 = jnp.sum(m1, axis=1, keepdims=True)
            cnt1 = jnp.maximum(s1, 1.0)
            cnt0 = jnp.maximum(jnp.float32(_C) - s1, 1.0)
            c0x = jnp.sum(px * m1, axis=1, keepdims=True) / cnt1
            c0y = jnp.sum(py * m1, axis=1, keepdims=True) / cnt1
            c1x = jnp.sum(px * m0, axis=1, keepdims=True) / cnt0
            c1y = jnp.sum(py * m0, axis=1, keepdims=True) / cnt0
        mask_ref[...] = m1

    @pl.when(i >= _PH)
    def _split():
        x = xs_ref[pl.ds((i - _PH) * _SPLIT_H, _SPLIT_H)]
        keep1 = (mask_ref[...] > 0.0)[None, None, :, :]
        c1_ref[...] = jnp.where(keep1, x, 0.0)
        c0_ref[...] = jnp.where(keep1, 0.0, x)


def kernel(feature_batch):
    xt = jnp.transpose(feature_batch, (1, 2, 0, 3))
    with jax.ensure_compile_time_eval():
        perm = jax.random.permutation(jax.random.key(1), _B)
        P = jax.nn.one_hot(perm, _B, dtype=jnp.float32)

    out_spec = pl.BlockSpec((_SPLIT_H, _W, _B, _C),
                            lambda i: (jnp.maximum(i - _PH, 0), 0, 0, 0))
    c0t, c1t = pl.pallas_call(
        _fused_body,
        grid=(_PH + _H // _SPLIT_H,),
        in_specs=[pl.BlockSpec((_IN_H, _W, _B, _C),
                               lambda i: (jnp.minimum(i, _PH - 1), 0, 0, 0)),
                  pl.BlockSpec((_B, _B), lambda i: (0, 0))],
        out_specs=[out_spec, out_spec],
        out_shape=[jax.ShapeDtypeStruct((_H, _W, _B, _C), jnp.float32),
                   jax.ShapeDtypeStruct((_H, _W, _B, _C), jnp.float32)],
        scratch_shapes=[pltpu.VMEM((_W, _B, _C), jnp.float32),
                        pltpu.VMEM((_B, _C), jnp.float32),
                        pltpu.VMEM((_B, _C), jnp.float32),
                        pltpu.VMEM((_B, _C), jnp.float32),
                        pltpu.VMEM((_H, _W, _B, _C), jnp.float32)],
    )(xt, P)
    return (jnp.transpose(c0t, (2, 0, 1, 3)), jnp.transpose(c1t, (2, 0, 1, 3)))


# speedup vs baseline: 1.1675x; 1.1675x over previous
import jax
import jax.numpy as jnp
from jax.experimental import pallas as pl
from jax.experimental.pallas import tpu as pltpu

_B, _H, _W, _C = 32, 14, 14, 512
_KM_ITERS = 11
_SPLIT_H = 2
_IN_H = 7
_PH = _H // _IN_H


def _fused_body(x_ref, perm_ref, c0_ref, c1_ref,
                cm_ref, bv_ref, bh_ref, mask_ref, xs_ref):
    i = pl.program_id(0)

    @pl.when(i == 0)
    def _init():
        cm_ref[...] = jnp.full((_W, _B, _C), -jnp.inf, jnp.float32)
        bv_ref[...] = jnp.full((_B, _C), -jnp.inf, jnp.float32)
        bh_ref[...] = jnp.zeros((_B, _C), jnp.float32)

    @pl.when(i < _PH)
    def _scan():
        x = x_ref[...]
        xs_ref[pl.ds(i * _IN_H, _IN_H)] = x
        cm_ref[...] = jnp.maximum(cm_ref[...], jnp.max(x, axis=0))
        rm = jnp.max(x, axis=1)
        bv = bv_ref[...]
        bh = bh_ref[...]
        for k in range(_IN_H):
            upd = rm[k] > bv
            bv = jnp.where(upd, rm[k], bv)
            bh = jnp.where(upd, (i * _IN_H + k).astype(jnp.float32), bh)
        bv_ref[...] = bv
        bh_ref[...] = bh

    @pl.when(i == _PH - 1)
    def _kmeans():
        cm = cm_ref[...]
        best = cm[0]
        aw = jnp.zeros((_B, _C), jnp.float32)
        for w in range(1, _W):
            upd = cm[w] > best
            best = jnp.where(upd, cm[w], best)
            aw = jnp.where(upd, jnp.float32(w), aw)
        px = aw
        py = bh_ref[...]

        P = perm_ref[...]
        cx = jnp.dot(P, px[:, 0:2], preferred_element_type=jnp.float32)
        cy = jnp.dot(P, py[:, 0:2], preferred_element_type=jnp.float32)
        c0x, c1x = cx[:, 0:1], cx[:, 1:2]
        c0y, c1y = cy[:, 0:1], cy[:, 1:2]
        m1 = jnp.zeros((_B, _C), jnp.float32)
        for _ in range(_KM_ITERS):
            d0 = (px - c0x) ** 2 + (py - c0y) ** 2
            d1 = (px - c1x) ** 2 + (py - c1y) ** 2
            m1 = (d1 < d0).astype(jnp.float32)
            m0 = 1.0 - m1
            s1 = jnp.sum(m1, axis=1, keepdims=True)
            cnt1 = jnp.maximum(s1, 1.0)
            cnt0 = jnp.maximum(jnp.float32(_C) - s1, 1.0)
            c0x = jnp.sum(px * m1, axis=1, keepdims=True) / cnt1
            c0y = jnp.sum(py * m1, axis=1, keepdims=True) / cnt1
            c1x = jnp.sum(px * m0, axis=1, keepdims=True) / cnt0
            c1y = jnp.sum(py * m0, axis=1, keepdims=True) / cnt0
        mask_ref[...] = m1

    @pl.when(i >= _PH)
    def _split():
        x = xs_ref[pl.ds((i - _PH) * _SPLIT_H, _SPLIT_H)]
        keep1 = (mask_ref[...] > 0.0)[None, None, :, :]
        c1_ref[...] = jnp.where(keep1, x, 0.0)
        c0_ref[...] = jnp.where(keep1, 0.0, x)


def kernel(feature_batch):
    xt = jnp.transpose(feature_batch, (1, 2, 0, 3))
    with jax.ensure_compile_time_eval():
        perm = jax.random.permutation(jax.random.key(1), _B)
        P = jax.nn.one_hot(perm, _B, dtype=jnp.float32)

    out_spec = pl.BlockSpec((_SPLIT_H, _W, _B, _C),
                            lambda i: (jnp.maximum(i - _PH, 0), 0, 0, 0))
    c0t, c1t = pl.pallas_call(
        _fused_body,
        grid=(_PH + _H // _SPLIT_H,),
        in_specs=[pl.BlockSpec((_IN_H, _W, _B, _C),
                               lambda i: (jnp.minimum(i, _PH - 1), 0, 0, 0)),
                  pl.BlockSpec((_B, _B), lambda i: (0, 0))],
        out_specs=[out_spec, out_spec],
        out_shape=[jax.ShapeDtypeStruct((_H, _W, _B, _C), jnp.float32),
                   jax.ShapeDtypeStruct((_H, _W, _B, _C), jnp.float32)],
        scratch_shapes=[pltpu.VMEM((_W, _B, _C), jnp.float32),
                        pltpu.VMEM((_B, _C), jnp.float32),
                        pltpu.VMEM((_B, _C), jnp.float32),
                        pltpu.VMEM((_B, _C), jnp.float32),
                        pltpu.VMEM((_H, _W, _B, _C), jnp.float32)],
    )(xt, P)
    return (jnp.transpose(c0t, (2, 0, 1, 3)), jnp.transpose(c1t, (2, 0, 1, 3)))
